# baseline (speedup 1.0000x reference)
import functools

import jax
import jax.numpy as jnp
from jax import lax
from jax.experimental import pallas as pl
from jax.experimental.pallas import tpu as pltpu
from jax.experimental.pallas import tpu_sc as plsc

N_NODES = 10000
N_EDGES = 320000
D = 128
HID = 128

NC = 2
NS = 16
NW = NC * NS
EPT = N_EDGES // NW
CH = 128
NCH = 80
NHALF = 2
CPH = NCH // NHALF
EPH = CPH * CH
NPAIR = CPH // 2
EPT_P = NCH * CH
N_PAD = 10240
RPT = N_PAD // NS
WCH = RPT // CH

_mesh = plsc.VectorSubcoreMesh(core_axis_name="c", subcore_axis_name="s")


@functools.partial(
    pl.kernel,
    out_type=(
        jax.ShapeDtypeStruct((NC * N_PAD, D), jnp.float32),
        jax.ShapeDtypeStruct((NC * N_PAD, D), jnp.float32),
    ),
    mesh=_mesh,
    scratch_types=[
        pltpu.VMEM((EPH,), jnp.int32),
        pltpu.VMEM((CPH, CH), jnp.int32),
        pltpu.VMEM((CH, D), jnp.float32),
        pltpu.VMEM((CH, D), jnp.float32),
        pltpu.VMEM_SHARED((N_PAD, D), jnp.float32),
        pltpu.SemaphoreType.DMA,
        pltpu.SemaphoreType.DMA,
        pltpu.SemaphoreType.DMA,
        pltpu.SemaphoreType.DMA,
    ],
)
def _sc_aggregate(src_hbm, tgt3_hbm, nf_hbm, zrows_hbm, ones_hbm,
                  psum_hbm, pcnt_hbm,
                  src_v, tgt_v, rows0, rows1, acc, g0, g1, s0, s1):
    c = lax.axis_index("c")
    s = lax.axis_index("s")
    wid = c * NS + s
    rbase = s * RPT
    obase = c * N_PAD + rbase

    def gather(j, buf, sem):
        off = pl.multiple_of(j * CH, 8)
        return pltpu.async_copy(nf_hbm.at[src_v.at[pl.ds(off, CH)]], buf, sem)

    def gwait(buf, sem):
        pltpu.make_async_copy(nf_hbm.at[src_v.at[pl.ds(0, CH)]], buf,
                              sem).wait()

    def scat(j, buf, sem):
        pltpu.async_copy(buf, acc.at[tgt_v.at[j]], sem, add=True)

    def swait(buf, sem):
        pltpu.make_async_copy(buf, acc.at[tgt_v.at[0]], sem).wait()

    pltpu.sync_copy(zrows_hbm, rows0)

    def zero_chunk(k, carry):
        roff = pl.multiple_of(rbase + k * CH, 8)
        pltpu.sync_copy(rows0, acc.at[pl.ds(roff, CH)])
        return carry

    lax.fori_loop(0, WCH, zero_chunk, 0)
    plsc.subcore_barrier()

    for h in range(NHALF):
        pltpu.sync_copy(src_hbm.at[pl.ds(wid * EPT_P + h * EPH, EPH)], src_v)
        pltpu.sync_copy(tgt3_hbm.at[pl.ds(wid * NCH + h * CPH, CPH)], tgt_v)
        gather(0, rows0, g0)
        gather(1, rows1, g1)

        def sum_pair(jj, carry):
            j0 = jj * 2
            j1 = j0 + 1
            gwait(rows0, g0)
            scat(j0, rows0, s0)
            gwait(rows1, g1)
            scat(j1, rows1, s1)

            @pl.when(jj < NPAIR - 1)
            def _():
                swait(rows0, s0)
                gather(j0 + 2, rows0, g0)
                swait(rows1, s1)
                gather(j1 + 2, rows1, g1)

            return carry

        lax.fori_loop(0, NPAIR, sum_pair, 0)
        swait(rows0, s0)
        swait(rows1, s1)

    plsc.subcore_barrier()

    def write_sum(k, carry):
        roff = pl.multiple_of(rbase + k * CH, 8)
        ooff = pl.multiple_of(obase + k * CH, 8)
        pltpu.sync_copy(acc.at[pl.ds(roff, CH)], rows0)
        pltpu.sync_copy(rows0, psum_hbm.at[pl.ds(ooff, CH)])
        return carry

    lax.fori_loop(0, WCH, write_sum, 0)
    plsc.subcore_barrier()

    pltpu.sync_copy(zrows_hbm, rows0)

    def zero_chunk2(k, carry):
        roff = pl.multiple_of(rbase + k * CH, 8)
        pltpu.sync_copy(rows0, acc.at[pl.ds(roff, CH)])
        return carry

    lax.fori_loop(0, WCH, zero_chunk2, 0)
    pltpu.sync_copy(ones_hbm, rows0)
    plsc.subcore_barrier()

    for h in range(NHALF):
        pltpu.sync_copy(tgt3_hbm.at[pl.ds(wid * NCH + h * CPH, CPH)], tgt_v)

        def cnt_pair(jj, carry):
            j0 = jj * 2
            j1 = j0 + 1

            @pl.when(jj > 0)
            def _():
                swait(rows0, s0)

            scat(j0, rows0, s0)

            @pl.when(jj > 0)
            def _():
                swait(rows0, s1)

            scat(j1, rows0, s1)
            return carry

        lax.fori_loop(0, NPAIR, cnt_pair, 0)
        swait(rows0, s0)
        swait(rows0, s1)

    plsc.subcore_barrier()

    def write_cnt(k, carry):
        roff = pl.multiple_of(rbase + k * CH, 8)
        ooff = pl.multiple_of(obase + k * CH, 8)
        pltpu.sync_copy(acc.at[pl.ds(roff, CH)], rows1)
        pltpu.sync_copy(rows1, pcnt_hbm.at[pl.ds(ooff, CH)])
        return carry

    lax.fori_loop(0, WCH, write_cnt, 0)


def _tc_mlp_body(p0, p1, c0, c1, nf, w1a, w1b, b1, w2, b2, out):
    inv = 1.0 / jnp.maximum(c0[:, 0:1] + c1[:, 0:1], 1.0)
    agg = (p0[...] + p1[...]) * inv
    h = (
        lax.dot_general(agg, w1a[...], (((1,), (1,)), ((), ())),
                        preferred_element_type=jnp.float32)
        + lax.dot_general(nf[...], w1b[...], (((1,), (1,)), ((), ())),
                          preferred_element_type=jnp.float32)
        + b1[...]
    )
    h = jnp.maximum(h, 0.0)
    out[...] = (
        lax.dot_general(h, w2[...], (((1,), (1,)), ((), ())),
                        preferred_element_type=jnp.float32)
        + b2[...]
    )


def kernel(node_features, edge_index, W1, b1, W2, b2):
    src = edge_index[0].astype(jnp.int32).reshape(NW, EPT)
    src = jnp.pad(src, ((0, 0), (0, EPT_P - EPT))).reshape(-1)
    tgt = edge_index[1].astype(jnp.int32).reshape(NW, EPT)
    tgt = jnp.pad(tgt, ((0, 0), (0, EPT_P - EPT)),
                  constant_values=N_PAD - 1).reshape(NW * NCH, CH)
    zrows = jnp.zeros((CH, D), jnp.float32)
    ones = jnp.ones((CH, D), jnp.float32)

    psum, pcnt = _sc_aggregate(src, tgt, node_features, zrows, ones)

    R = 1000
    grid = (N_NODES // R,)
    out = pl.pallas_call(
        _tc_mlp_body,
        grid=grid,
        in_specs=[
            pl.BlockSpec((R, D), lambda i: (i, 0)),
            pl.BlockSpec((R, D), lambda i: (i, 0)),
            pl.BlockSpec((R, D), lambda i: (i, 0)),
            pl.BlockSpec((R, D), lambda i: (i, 0)),
            pl.BlockSpec((R, D), lambda i: (i, 0)),
            pl.BlockSpec((HID, D), lambda i: (0, 0)),
            pl.BlockSpec((HID, D), lambda i: (0, 0)),
            pl.BlockSpec((1, HID), lambda i: (0, 0)),
            pl.BlockSpec((D, HID), lambda i: (0, 0)),
            pl.BlockSpec((1, D), lambda i: (0, 0)),
        ],
        out_specs=pl.BlockSpec((R, D), lambda i: (i, 0)),
        out_shape=jax.ShapeDtypeStruct((N_NODES, D), jnp.float32),
    )(
        psum[:N_NODES], psum[N_PAD:N_PAD + N_NODES],
        pcnt[:N_NODES], pcnt[N_PAD:N_PAD + N_NODES],
        node_features,
        W1[:, :D], W1[:, D:],
        b1.reshape(1, HID), W2, b2.reshape(1, D),
    )
    return out
